# baseline (speedup 1.0000x reference)
import jax
import jax.numpy as jnp
from jax.experimental import pallas as pl

_B, _J, _D = 1024, 66, 240
_NUP = 42
_BBLK = 128


_RDJ = 48


def _body(x_ref, out_ref, masked_ref):
    x = x_ref[...][:, :_NUP, :]
    out_ref[...] = x
    masked_ref[...] = jnp.concatenate(
        [x, jnp.zeros((x.shape[0], _J - _NUP, _D), x.dtype)], axis=1
    )


def kernel(input):
    out, masked = pl.pallas_call(
        _body,
        grid=(_B // _BBLK,),
        in_specs=[pl.BlockSpec((_BBLK, _RDJ, _D), lambda i: (i, 0, 0))],
        out_specs=[
            pl.BlockSpec((_BBLK, _NUP, _D), lambda i: (i, 0, 0)),
            pl.BlockSpec((_BBLK, _J, _D), lambda i: (i, 0, 0)),
        ],
        out_shape=[
            jax.ShapeDtypeStruct((_B, _NUP, _D), input.dtype),
            jax.ShapeDtypeStruct((_B, _J, _D), input.dtype),
        ],
    )(input)
    return (out, masked)
